# baseline (speedup 1.0000x reference)
import functools
import jax
import jax.numpy as jnp
from jax import lax
from jax.experimental import pallas as pl
from jax.experimental.pallas import tpu as pltpu
from jax.experimental.pallas import tpu_sc as plsc

LANES = 16
N_CORES = 2
N_SUBCORES = 16


@jax.jit
def _run(user_ids, movie_ids, uf_t, mf_t):
    B = user_ids.shape[0]
    F = uf_t.shape[0]
    NW = N_CORES * N_SUBCORES
    BPW = B // NW

    mesh = plsc.VectorSubcoreMesh(core_axis_name="c", subcore_axis_name="s")

    @functools.partial(
        pl.kernel,
        mesh=mesh,
        compiler_params=pltpu.CompilerParams(use_tc_tiling_on_sc=False),
        out_type=jax.ShapeDtypeStruct((B,), jnp.float32),
        scratch_types=[
            pltpu.VMEM((BPW,), jnp.int32),
            pltpu.VMEM((BPW,), jnp.int32),
            pltpu.VMEM((BPW,), jnp.float32),
            pltpu.VMEM((BPW,), jnp.float32),
            pltpu.VMEM((BPW,), jnp.float32),
            pltpu.SemaphoreType.DMA,
            pltpu.SemaphoreType.DMA,
        ],
    )
    def sc_kernel(uids_hbm, mids_hbm, uf_hbm, mf_hbm, out_hbm,
                  uidx_v, midx_v, ubuf_v, mbuf_v, out_v, sem_u, sem_m):
        wid = lax.axis_index("s") * N_CORES + lax.axis_index("c")
        base = wid * BPW

        pltpu.sync_copy(uids_hbm.at[pl.ds(base, BPW)], uidx_v)
        pltpu.sync_copy(mids_hbm.at[pl.ds(base, BPW)], midx_v)

        def fstep(f, carry):
            cp_u = pltpu.async_copy(uf_hbm.at[f].at[uidx_v], ubuf_v, sem_u)
            cp_m = pltpu.async_copy(mf_hbm.at[f].at[midx_v], mbuf_v, sem_m)
            cp_u.wait()
            cp_m.wait()

            def group(g, c2):
                u = ubuf_v[pl.ds(g * LANES, LANES)]
                m = mbuf_v[pl.ds(g * LANES, LANES)]
                o = out_v[pl.ds(g * LANES, LANES)]
                out_v[pl.ds(g * LANES, LANES)] = o + u * m
                return c2

            lax.fori_loop(0, BPW // LANES, group, 0)
            return carry

        def zgroup(g, c2):
            out_v[pl.ds(g * LANES, LANES)] = jnp.zeros((LANES,), jnp.float32)
            return c2

        lax.fori_loop(0, BPW // LANES, zgroup, 0)
        lax.fori_loop(0, F, fstep, 0)

        pltpu.sync_copy(out_v, out_hbm.at[pl.ds(base, BPW)])

    return sc_kernel(user_ids, movie_ids, uf_t, mf_t)


def kernel(user_ids, movie_ids, user_factors, movie_factors):
    out = _run(user_ids.astype(jnp.int32), movie_ids.astype(jnp.int32),
               user_factors.T, movie_factors.T)
    return out.reshape(-1, 1)


# speedup vs baseline: 18.5406x; 18.5406x over previous
import functools
import jax
import jax.numpy as jnp
from jax import lax
from jax.experimental import pallas as pl
from jax.experimental.pallas import tpu as pltpu
from jax.experimental.pallas import tpu_sc as plsc

LANES = 16
N_CORES = 2
N_SUBCORES = 16
GRP = 8
TCW = 128
LAST_BASE = 0


@jax.jit
def _run(user_ids, movie_ids, uf_t, mf_t):
    B = user_ids.shape[0]
    F = uf_t.shape[0]
    V = uf_t.shape[1]
    NW = N_CORES * N_SUBCORES
    BPW = B // NW

    n_full = (V // TCW) * TCW
    last_base = n_full - TCW
    tail_w = V - n_full

    mesh = plsc.VectorSubcoreMesh(core_axis_name="c", subcore_axis_name="s")

    @functools.partial(
        pl.kernel,
        mesh=mesh,
        compiler_params=pltpu.CompilerParams(needs_layout_passes=False),
        out_type=jax.ShapeDtypeStruct((B,), jnp.float32),
        scratch_types=[
            pltpu.VMEM((BPW,), jnp.int32),
            pltpu.VMEM((BPW,), jnp.int32),
            pltpu.VMEM((GRP, F, TCW), jnp.float32),
            pltpu.VMEM((GRP, F, TCW), jnp.float32),
            pltpu.VMEM((F, tail_w), jnp.float32),
            pltpu.VMEM((F, tail_w), jnp.float32),
            pltpu.VMEM((BPW,), jnp.float32),
            pltpu.SemaphoreType.DMA,
            pltpu.SemaphoreType.DMA,
        ],
    )
    def sc_kernel(uids_hbm, mids_hbm, uf_hbm, mf_hbm, out_hbm,
                  uidx_v, midx_v, uring_v, mring_v, utail_v, mtail_v,
                  out_v, sem_u, sem_m):
        wid = lax.axis_index("s") * N_CORES + lax.axis_index("c")
        base = wid * BPW

        pltpu.sync_copy(uids_hbm.at[pl.ds(base, BPW)], uidx_v)
        pltpu.sync_copy(mids_hbm.at[pl.ds(base, BPW)], midx_v)
        pltpu.sync_copy(uf_hbm.at[:, pl.ds(n_full, tail_w)], utail_v)
        pltpu.sync_copy(mf_hbm.at[:, pl.ds(n_full, tail_w)], mtail_v)

        lane = lax.broadcasted_iota(jnp.int32, (LANES,), 0)
        slot = jnp.bitwise_and(lane, GRP - 1)

        def pair(p, carry):
            uvec = uidx_v[pl.ds(p * 2 * GRP, LANES)]
            mvec = midx_v[pl.ds(p * 2 * GRP, LANES)]
            ubase = jnp.minimum(
                jnp.bitwise_and(uvec, ~(TCW - 1)), last_base)
            mbase = jnp.minimum(
                jnp.bitwise_and(mvec, ~(TCW - 1)), last_base)
            uc = uvec - ubase
            mc = mvec - mbase
            ucl = jnp.minimum(uc, TCW - 1)
            mcl = jnp.minimum(mc, TCW - 1)
            uct = jnp.bitwise_and(uvec - n_full, tail_w - 1)
            mct = jnp.bitwise_and(mvec - n_full, tail_w - 1)
            u_is_tail = uvec >= n_full
            m_is_tail = mvec >= n_full

            def phase(lo):
                cps = []
                for j in range(GRP):
                    cps.append(pltpu.async_copy(
                        uf_hbm.at[:, pl.ds(pl.multiple_of(ubase[lo + j], TCW), TCW)],
                        uring_v.at[j], sem_u))
                    cps.append(pltpu.async_copy(
                        mf_hbm.at[:, pl.ds(pl.multiple_of(mbase[lo + j], TCW), TCW)],
                        mring_v.at[j], sem_m))
                for cp in cps:
                    cp.wait()
                acc = jnp.zeros((LANES,), jnp.float32)
                for f in range(F):
                    fvec = jnp.full((LANES,), f, jnp.int32)
                    u = plsc.load_gather(uring_v, [slot, fvec, ucl])
                    m = plsc.load_gather(mring_v, [slot, fvec, mcl])
                    ut = plsc.load_gather(utail_v, [fvec, uct])
                    mt = plsc.load_gather(mtail_v, [fvec, mct])
                    uv = jnp.where(u_is_tail, ut, u)
                    mv = jnp.where(m_is_tail, mt, m)
                    acc = acc + uv * mv
                return acc

            acc_lo = phase(0)
            acc_hi = phase(GRP)
            res = jnp.where(lane < GRP, acc_lo, acc_hi)
            out_v[pl.ds(p * 2 * GRP, LANES)] = res
            return carry

        lax.fori_loop(0, BPW // (2 * GRP), pair, 0)

        pltpu.sync_copy(out_v, out_hbm.at[pl.ds(base, BPW)])

    return sc_kernel(user_ids, movie_ids, uf_t, mf_t)


def kernel(user_ids, movie_ids, user_factors, movie_factors):
    out = _run(user_ids.astype(jnp.int32), movie_ids.astype(jnp.int32),
               user_factors.T, movie_factors.T)
    return out.reshape(-1, 1)
